# Initial kernel scaffold; baseline (speedup 1.0000x reference)
import math

import jax
import jax.numpy as jnp
from jax.experimental import pallas as pl

_ORI = 9
_H = 512
_W = 512
_CH = 8
_PH = _H // _CH
_PW = _W // _CH


def _hog_body(x_ref, o_ref):
    x = x_ref[0, 0]

    zr = jnp.zeros((1, _W), jnp.float32)
    zc = jnp.zeros((_H, 1), jnp.float32)

    def up(a):
        return jnp.concatenate([a[1:, :], zr], axis=0)

    def dn(a):
        return jnp.concatenate([zr, a[:-1, :]], axis=0)

    def lf(a):
        return jnp.concatenate([zc, a[:, :-1]], axis=1)

    def rt(a):
        return jnp.concatenate([a[:, 1:], zc], axis=1)

    sv = dn(x) + 2.0 * x + up(x)
    g0 = lf(sv) - rt(sv)
    sh = lf(x) + 2.0 * x + rt(x)
    g1 = dn(sh) - up(sh)

    mag = jnp.sqrt(jnp.maximum(g0 * g0 + g1 * g1, 1e-30))

    s = [None] * 10
    s[0] = g0
    for b in range(1, 9):
        ang = b * math.pi / _ORI
        s[b] = jnp.float32(math.cos(ang)) * g0 - jnp.float32(math.sin(ang)) * g1
    s[9] = -g0

    ge = [v >= 0 for v in s]
    gt = [v > 0 for v in s]
    f = [(ge[b] & ~ge[b + 1]) | (~gt[b] & gt[b + 1]) for b in range(_ORI)]

    bnd = g0 == 0.0

    r = jax.lax.broadcasted_iota(jnp.int32, (_H, _PW), 0)
    c = jax.lax.broadcasted_iota(jnp.int32, (_H, _PW), 1)
    P = (r // _CH == c).astype(jnp.float32)
    rT = jax.lax.broadcasted_iota(jnp.int32, (_PH, _W), 0)
    cT = jax.lax.broadcasted_iota(jnp.int32, (_PH, _W), 1)
    PT = (cT // _CH == rT).astype(jnp.float32)

    inv = jnp.float32(1.0 / (_CH * _CH))
    for b in range(_ORI):
        fb = f[b].astype(jnp.float32)
        cb = jnp.where(bnd, f[b], f[(b - 1) % _ORI]).astype(jnp.float32)
        wb = (fb + cb) * mag
        t = jnp.dot(PT, wb, preferred_element_type=jnp.float32)
        o_ref[0, b] = jnp.dot(t, P, preferred_element_type=jnp.float32) * inv


def kernel(x, weight):
    n = x.shape[0]
    pooled = pl.pallas_call(
        _hog_body,
        grid=(n,),
        in_specs=[pl.BlockSpec((1, 1, _H, _W), lambda i: (i, 0, 0, 0))],
        out_specs=pl.BlockSpec((1, _ORI, _PH, _PW), lambda i: (i, 0, 0, 0)),
        out_shape=jax.ShapeDtypeStruct((n, _ORI, _PH, _PW), jnp.float32),
    )(x)
    return pooled.reshape(n, -1)


# speedup vs baseline: 11.2877x; 11.2877x over previous
import math

import jax
import jax.numpy as jnp
from jax.experimental import pallas as pl

_ORI = 9
_H = 512
_W = 512
_CH = 8
_PH = _H // _CH
_PW = _W // _CH


def _hog_body(x_ref, o_ref):
    x = x_ref[0, 0].astype(jnp.bfloat16).astype(jnp.float32)

    zr = jnp.zeros((1, _W), jnp.float32)
    zc = jnp.zeros((_H, 1), jnp.float32)

    def up(a):
        return jnp.concatenate([a[1:, :], zr], axis=0)

    def dn(a):
        return jnp.concatenate([zr, a[:-1, :]], axis=0)

    def lf(a):
        return jnp.concatenate([zc, a[:, :-1]], axis=1)

    def rt(a):
        return jnp.concatenate([a[:, 1:], zc], axis=1)

    sv = dn(x) + 2.0 * x + up(x)
    g0 = lf(sv) - rt(sv)
    sh = lf(x) + 2.0 * x + rt(x)
    g1 = dn(sh) - up(sh)

    mag = jnp.sqrt(jnp.maximum(g0 * g0 + g1 * g1, 1e-30))

    s = [None] * 10
    s[0] = g0
    for b in range(1, 9):
        ang = b * math.pi / _ORI
        s[b] = jnp.float32(math.cos(ang)) * g0 - jnp.float32(math.sin(ang)) * g1
    s[9] = -g0

    one = jnp.float32(1.0)
    zero = jnp.float32(0.0)
    ge = [jnp.where(v >= 0, one, zero) for v in s]
    gt = [jnp.where(v > 0, one, zero) for v in s]
    f = [ge[b] * (one - ge[b + 1]) + (one - gt[b]) * gt[b + 1]
         for b in range(_ORI)]

    bnd = g0 == 0.0

    r = jax.lax.broadcasted_iota(jnp.int32, (_H, _PW), 0)
    c = jax.lax.broadcasted_iota(jnp.int32, (_H, _PW), 1)
    P = jnp.where(r // _CH == c, one, zero)
    rT = jax.lax.broadcasted_iota(jnp.int32, (_PH, _W), 0)
    cT = jax.lax.broadcasted_iota(jnp.int32, (_PH, _W), 1)
    PT = jnp.where(cT // _CH == rT, one, zero)

    inv = jnp.float32(1.0 / (_CH * _CH))
    for b in range(_ORI):
        fb = f[b]
        cb = jnp.where(bnd, f[b], f[(b - 1) % _ORI])
        wb = (fb + cb) * mag
        t = jnp.dot(PT, wb, preferred_element_type=jnp.float32,
                    precision=jax.lax.Precision.HIGHEST)
        o_ref[0, b] = jnp.dot(t, P, preferred_element_type=jnp.float32,
                              precision=jax.lax.Precision.HIGHEST) * inv


def kernel(x, weight):
    n = x.shape[0]
    pooled = pl.pallas_call(
        _hog_body,
        grid=(n,),
        in_specs=[pl.BlockSpec((1, 1, _H, _W), lambda i: (i, 0, 0, 0))],
        out_specs=pl.BlockSpec((1, _ORI, _PH, _PW), lambda i: (i, 0, 0, 0)),
        out_shape=jax.ShapeDtypeStruct((n, _ORI, _PH, _PW), jnp.float32),
    )(x)
    return pooled.reshape(n, -1)


# speedup vs baseline: 12.7200x; 1.1269x over previous
import math

import jax
import jax.numpy as jnp
from jax.experimental import pallas as pl

_ORI = 9
_H = 512
_W = 512
_CH = 8
_PH = _H // _CH
_PW = _W // _CH

_HIGH = jax.lax.Precision.HIGHEST


def _hog_body(x_ref, t_ref, d_ref, pt_ref, p_ref, o_ref):
    x = x_ref[0, 0].astype(jnp.bfloat16).astype(jnp.float32)
    T = t_ref[...]
    Dr = d_ref[...]
    PT = pt_ref[...]
    P = p_ref[...]

    zc = jnp.zeros((_H, 1), jnp.float32)

    def lf(a):
        return jnp.concatenate([zc, a[:, :-1]], axis=1)

    def rt(a):
        return jnp.concatenate([a[:, 1:], zc], axis=1)

    sv = jnp.dot(T, x, preferred_element_type=jnp.float32)
    g0 = lf(sv) - rt(sv)
    dv = jnp.dot(Dr, x, preferred_element_type=jnp.float32)
    g1 = lf(dv) + 2.0 * dv + rt(dv)

    mag = jnp.sqrt(jnp.maximum(g0 * g0 + g1 * g1, 1e-30))

    s = [None] * 10
    s[0] = g0
    for b in range(1, 9):
        ang = b * math.pi / _ORI
        s[b] = jnp.float32(math.cos(ang)) * g0 - jnp.float32(math.sin(ang)) * g1
    s[9] = -g0

    ge = [v >= 0 for v in s]
    f = [ge[b] != ge[b + 1] for b in range(_ORI)]

    bnd = g0 == 0.0
    zero = jnp.zeros_like(mag)
    u = [None] * _ORI
    u[0] = jnp.where(f[0] | bnd, mag, zero)
    for b in range(1, 8):
        u[b] = jnp.where(f[b], mag, zero)
    u[8] = jnp.where(f[8] & (~bnd), mag, zero)
    bz = jnp.where(bnd, mag, zero)

    R = [jnp.dot(PT, ub, preferred_element_type=jnp.float32, precision=_HIGH)
         for ub in u]
    Rz = jnp.dot(PT, bz, preferred_element_type=jnp.float32, precision=_HIGH)

    inv = jnp.float32(1.0 / (_CH * _CH))
    for b in range(_ORI):
        if b == 0:
            t = R[0] + R[8] + Rz
        elif b == 1:
            t = R[1] + R[0] - Rz
        else:
            t = R[b] + R[b - 1]
        o_ref[0, b] = jnp.dot(t, P, preferred_element_type=jnp.float32,
                              precision=_HIGH) * inv


def kernel(x, weight):
    n = x.shape[0]
    i = jnp.arange(_H, dtype=jnp.int32)
    d = i[:, None] - i[None, :]
    one = jnp.float32(1.0)
    zero = jnp.float32(0.0)
    T = (jnp.where(jnp.abs(d) == 1, one, zero)
         + jnp.where(d == 0, jnp.float32(2.0), zero))
    Dr = jnp.where(d == 1, one, zero) - jnp.where(d == -1, one, zero)
    pr = jnp.arange(_PH, dtype=jnp.int32)
    PT = jnp.where(i[None, :] // _CH == pr[:, None], one, zero)
    P = jnp.where(i[:, None] // _CH == pr[None, :], one, zero)

    pooled = pl.pallas_call(
        _hog_body,
        grid=(n,),
        in_specs=[
            pl.BlockSpec((1, 1, _H, _W), lambda i: (i, 0, 0, 0)),
            pl.BlockSpec((_H, _H), lambda i: (0, 0)),
            pl.BlockSpec((_H, _H), lambda i: (0, 0)),
            pl.BlockSpec((_PH, _H), lambda i: (0, 0)),
            pl.BlockSpec((_H, _PW), lambda i: (0, 0)),
        ],
        out_specs=pl.BlockSpec((1, _ORI, _PH, _PW), lambda i: (i, 0, 0, 0)),
        out_shape=jax.ShapeDtypeStruct((n, _ORI, _PH, _PW), jnp.float32),
    )(x, T, Dr, PT, P)
    return pooled.reshape(n, -1)


# speedup vs baseline: 20.0262x; 1.5744x over previous
import math

import jax
import jax.numpy as jnp
from jax.experimental import pallas as pl

_ORI = 9
_H = 512
_W = 512
_CH = 8
_PH = _H // _CH
_PW = _W // _CH

_HIGHEST = jax.lax.Precision.HIGHEST


def _hog_body(x_ref, t_ref, d_ref, pt_ref, p_ref, o_ref):
    x = x_ref[0, 0]
    T = t_ref[...]
    Dr = d_ref[...]
    PT = pt_ref[...]
    P = p_ref[...]

    zc = jnp.zeros((_H, 1), jnp.float32)

    def lf(a):
        return jnp.concatenate([zc, a[:, :-1]], axis=1)

    def rt(a):
        return jnp.concatenate([a[:, 1:], zc], axis=1)

    sv = jnp.dot(T, x, preferred_element_type=jnp.float32)
    g0 = lf(sv) - rt(sv)
    dv = jnp.dot(Dr, x, preferred_element_type=jnp.float32)
    g1 = lf(dv) + 2.0 * dv + rt(dv)

    mag = jnp.sqrt(jnp.maximum(g0 * g0 + g1 * g1, 1e-30))

    s = [None] * 10
    s[0] = g0
    for b in range(1, 9):
        ang = b * math.pi / _ORI
        s[b] = jnp.float32(math.cos(ang)) * g0 - jnp.float32(math.sin(ang)) * g1
    s[9] = -g0

    ge = [v >= 0 for v in s]
    f = [ge[b] != ge[b + 1] for b in range(_ORI)]

    bnd = g0 == 0.0
    zero = jnp.zeros_like(mag)

    mh = mag.astype(jnp.bfloat16).astype(jnp.float32)
    ml = mag - mh

    masks = [f[0] | bnd] + [f[b] for b in range(1, 8)] + [f[8] & (~bnd), bnd]

    def rowpool(m):
        uh = jnp.where(m, mh, zero)
        ul = jnp.where(m, ml, zero)
        return (jnp.dot(PT, uh, preferred_element_type=jnp.float32)
                + jnp.dot(PT, ul, preferred_element_type=jnp.float32))

    pools = [rowpool(m) for m in masks]
    R = pools[:_ORI]
    Rz = pools[_ORI]

    inv = jnp.float32(1.0 / (_CH * _CH))
    for b in range(_ORI):
        if b == 0:
            t = R[0] + R[8] + Rz
        elif b == 1:
            t = R[1] + R[0] - Rz
        else:
            t = R[b] + R[b - 1]
        o_ref[0, b] = jnp.dot(t, P, preferred_element_type=jnp.float32,
                              precision=_HIGHEST) * inv


def kernel(x, weight):
    n = x.shape[0]
    i = jnp.arange(_H, dtype=jnp.int32)
    d = i[:, None] - i[None, :]
    one = jnp.float32(1.0)
    zero = jnp.float32(0.0)
    T = (jnp.where(jnp.abs(d) == 1, one, zero)
         + jnp.where(d == 0, jnp.float32(2.0), zero))
    Dr = jnp.where(d == 1, one, zero) - jnp.where(d == -1, one, zero)
    pr = jnp.arange(_PH, dtype=jnp.int32)
    PT = jnp.where(i[None, :] // _CH == pr[:, None], one, zero)
    P = jnp.where(i[:, None] // _CH == pr[None, :], one, zero)

    pooled = pl.pallas_call(
        _hog_body,
        grid=(n,),
        in_specs=[
            pl.BlockSpec((1, 1, _H, _W), lambda i: (i, 0, 0, 0)),
            pl.BlockSpec((_H, _H), lambda i: (0, 0)),
            pl.BlockSpec((_H, _H), lambda i: (0, 0)),
            pl.BlockSpec((_PH, _H), lambda i: (0, 0)),
            pl.BlockSpec((_H, _PW), lambda i: (0, 0)),
        ],
        out_specs=pl.BlockSpec((1, _ORI, _PH, _PW), lambda i: (i, 0, 0, 0)),
        out_shape=jax.ShapeDtypeStruct((n, _ORI, _PH, _PW), jnp.float32),
    )(x, T, Dr, PT, P)
    return pooled.reshape(n, -1)
